# baseline (speedup 1.0000x reference)
import functools

import jax
import jax.numpy as jnp
from jax import lax
from jax.experimental import pallas as pl
from jax.experimental.pallas import tpu as pltpu
from jax.experimental.pallas import tpu_sc as plsc

N_NODES = 10000
D = 128
N_EDGES = 320000

NC = 2
NS = 16
NW = NC * NS

CHUNK = 64
KCH = 162
CPW = KCH * CHUNK
E_PAD = NW * CPW
N1 = 10240
ROWS_PT = N1 // NS
DEGW = 128
DCHUNK = 128
DKCH = CPW // DCHUNK

_mesh = plsc.VectorSubcoreMesh(core_axis_name="c", subcore_axis_name="s")


@functools.partial(
    pl.kernel,
    out_type=jax.ShapeDtypeStruct((NC * N1, DEGW), jnp.float32),
    mesh=_mesh,
    scratch_types=[
        pltpu.VMEM((DCHUNK,), jnp.int32),
        pltpu.VMEM((DCHUNK,), jnp.int32),
        pltpu.VMEM((DCHUNK,), jnp.int32),
        pltpu.VMEM((DCHUNK, DEGW), jnp.float32),
        pltpu.VMEM_SHARED((N1, DEGW), jnp.float32),
        pltpu.SemaphoreType.DMA,
        pltpu.SemaphoreType.DMA,
        pltpu.SemaphoreType.DMA,
        pltpu.SemaphoreType.DMA,
        pltpu.SemaphoreType.DMA,
        pltpu.SemaphoreType.DMA,
    ],
)
def _sc_degree(dst_hbm, ones_hbm, zeros_hbm, out_hbm,
               idxd0, idxd1, idxd2, ones_v, deg_sh,
               semi0, semi1, semi2, semsc0, semsc1, semsc2):
    c = lax.axis_index("c")
    s = lax.axis_index("s")
    base = (c * NS + s) * CPW
    idxd = (idxd0, idxd1, idxd2)
    semi = (semi0, semi1, semi2)
    semsc = (semsc0, semsc1, semsc2)

    def idx_load(j, sl):
        pltpu.async_copy(
            dst_hbm.at[pl.ds(base + j * DCHUNK, DCHUNK)], idxd[sl], semi[sl])

    def idx_wait(j, sl):
        pltpu.make_async_copy(
            dst_hbm.at[pl.ds(base + j * DCHUNK, DCHUNK)], idxd[sl], semi[sl]).wait()

    def sc_issue(sl):
        pltpu.async_copy(ones_v, deg_sh.at[idxd[sl]], semsc[sl], add=True)

    def sc_wait(sl):
        pltpu.make_async_copy(ones_v, deg_sh.at[idxd[sl]], semsc[sl]).wait()

    pltpu.sync_copy(ones_hbm, ones_v)
    idx_load(0, 0)
    idx_load(1, 1)
    pltpu.sync_copy(zeros_hbm, deg_sh.at[pl.ds(s * ROWS_PT, ROWS_PT)])
    plsc.subcore_barrier()

    def body(g, carry):
        for k in range(3):
            j = 3 * g + k
            idx_wait(j, k)
            sc_issue(k)

            @pl.when(j >= 1)
            def _():
                sc_wait((k + 2) % 3)

            @pl.when(j + 2 < DKCH)
            def _():
                idx_load(j + 2, (k + 2) % 3)
        return carry

    lax.fori_loop(0, DKCH // 3, body, 0)
    sc_wait(2)
    plsc.subcore_barrier()
    pltpu.sync_copy(
        deg_sh.at[pl.ds(s * ROWS_PT, ROWS_PT)],
        out_hbm.at[pl.ds(c * N1 + s * ROWS_PT, ROWS_PT)],
    )


@functools.partial(
    pl.kernel,
    out_type=jax.ShapeDtypeStruct((NC * N1, D), jnp.float32),
    mesh=_mesh,
    scratch_types=[
        pltpu.VMEM((CHUNK,), jnp.int32),
        pltpu.VMEM((CHUNK,), jnp.int32),
        pltpu.VMEM((CHUNK,), jnp.int32),
        pltpu.VMEM((CHUNK,), jnp.int32),
        pltpu.VMEM((CHUNK,), jnp.int32),
        pltpu.VMEM((CHUNK,), jnp.int32),
        pltpu.VMEM((CHUNK,), jnp.int32),
        pltpu.VMEM((CHUNK,), jnp.int32),
        pltpu.VMEM((CHUNK,), jnp.int32),
        pltpu.VMEM((CHUNK,), jnp.int32),
        pltpu.VMEM((CHUNK,), jnp.int32),
        pltpu.VMEM((CHUNK,), jnp.int32),
        pltpu.VMEM((CHUNK, D), jnp.float32),
        pltpu.VMEM((CHUNK, D), jnp.float32),
        pltpu.VMEM((CHUNK, D), jnp.float32),
        pltpu.VMEM_SHARED((N1, D), jnp.float32),
        pltpu.SemaphoreType.DMA,
        pltpu.SemaphoreType.DMA,
        pltpu.SemaphoreType.DMA,
        pltpu.SemaphoreType.DMA,
        pltpu.SemaphoreType.DMA,
        pltpu.SemaphoreType.DMA,
        pltpu.SemaphoreType.DMA,
        pltpu.SemaphoreType.DMA,
        pltpu.SemaphoreType.DMA,
        pltpu.SemaphoreType.DMA,
        pltpu.SemaphoreType.DMA,
        pltpu.SemaphoreType.DMA,
    ],
)
def _sc_aggregate(g_hbm, src_hbm, dst_hbm, zeros_hbm, out_hbm,
                  idxs0, idxs1, idxs2, idxs3, idxs4, idxs5,
                  idxd0, idxd1, idxd2, idxd3, idxd4, idxd5,
                  rows0, rows1, rows2, agg_sh,
                  semi0, semi1, semi2, semi3, semi4, semi5,
                  semg0, semg1, semg2, semsc0, semsc1, semsc2):
    c = lax.axis_index("c")
    s = lax.axis_index("s")
    base = (c * NS + s) * CPW
    idxs = (idxs0, idxs1, idxs2, idxs3, idxs4, idxs5)
    idxd = (idxd0, idxd1, idxd2, idxd3, idxd4, idxd5)
    rows = (rows0, rows1, rows2)
    semi = (semi0, semi1, semi2, semi3, semi4, semi5)
    semg = (semg0, semg1, semg2)
    semsc = (semsc0, semsc1, semsc2)

    def idx_load(j, sl):
        pltpu.async_copy(
            src_hbm.at[pl.ds(base + j * CHUNK, CHUNK)], idxs[sl], semi[sl])
        pltpu.async_copy(
            dst_hbm.at[pl.ds(base + j * CHUNK, CHUNK)], idxd[sl], semi[sl])

    def idx_wait(j, sl):
        pltpu.make_async_copy(
            src_hbm.at[pl.ds(base + j * CHUNK, CHUNK)], idxs[sl], semi[sl]).wait()
        pltpu.make_async_copy(
            dst_hbm.at[pl.ds(base + j * CHUNK, CHUNK)], idxd[sl], semi[sl]).wait()

    def gather(sl, r):
        pltpu.async_copy(g_hbm.at[idxs[sl]], rows[r], semg[r])

    def gather_wait(sl, r):
        pltpu.make_async_copy(g_hbm.at[idxs[sl]], rows[r], semg[r]).wait()

    def sc_issue(sl, r):
        pltpu.async_copy(rows[r], agg_sh.at[idxd[sl]], semsc[r], add=True)

    def sc_wait(sl, r):
        pltpu.make_async_copy(rows[r], agg_sh.at[idxd[sl]], semsc[r]).wait()

    for k in range(5):
        idx_load(k, k)
    pltpu.sync_copy(zeros_hbm, agg_sh.at[pl.ds(s * ROWS_PT, ROWS_PT)])
    plsc.subcore_barrier()
    idx_wait(0, 0)
    gather(0, 0)
    idx_wait(1, 1)
    gather(1, 1)

    def body(g, carry):
        for k in range(6):
            j = 6 * g + k
            r = k % 3
            rm1 = (k + 2) % 3
            gather_wait(k, r)
            sc_issue(k, r)

            @pl.when(j >= 1)
            def _():
                sc_wait((k + 5) % 6, rm1)

            @pl.when(j + 5 < KCH)
            def _():
                idx_load(j + 5, (k + 5) % 6)

            @pl.when(j + 2 < KCH)
            def _():
                idx_wait(j + 2, (k + 2) % 6)
                gather((k + 2) % 6, rm1)
        return carry

    lax.fori_loop(0, KCH // 6, body, 0)
    sc_wait(5, 2)
    plsc.subcore_barrier()
    pltpu.sync_copy(
        agg_sh.at[pl.ds(s * ROWS_PT, ROWS_PT)],
        out_hbm.at[pl.ds(c * N1 + s * ROWS_PT, ROWS_PT)],
    )


BR = 1024


def _tc1_body(degp0, degp1, x_ref, w_ref, dinv_ref, g_ref):
    deg = degp0[:, :1] + degp1[:, :1] + 1.0
    dinvb = jnp.broadcast_to(lax.rsqrt(deg), (BR, D))
    h = jnp.dot(x_ref[...], w_ref[...], preferred_element_type=jnp.float32)
    dinv_ref[...] = dinvb
    g_ref[...] = dinvb * h


_tc1 = pl.pallas_call(
    _tc1_body,
    grid=(N1 // BR,),
    in_specs=[
        pl.BlockSpec((BR, DEGW), lambda i: (i, 0)),
        pl.BlockSpec((BR, DEGW), lambda i: (i, 0)),
        pl.BlockSpec((BR, D), lambda i: (i, 0)),
        pl.BlockSpec((D, D), lambda i: (0, 0)),
    ],
    out_specs=[
        pl.BlockSpec((BR, D), lambda i: (i, 0)),
        pl.BlockSpec((BR, D), lambda i: (i, 0)),
    ],
    out_shape=[
        jax.ShapeDtypeStruct((N1, D), jnp.float32),
        jax.ShapeDtypeStruct((N1, D), jnp.float32),
    ],
)


def _tc2_body(agg0, agg1, g1, dinv, w_ref, b_ref, g2_ref):
    i = pl.program_id(0)
    t = dinv[...] * (agg0[...] + agg1[...] + g1[...]) + b_ref[...]
    t = jnp.maximum(t, 0.0)
    rows = i * BR + lax.broadcasted_iota(jnp.int32, (BR, D), 0)
    t = jnp.where(rows < N_NODES, t, 0.0)
    g2_ref[...] = dinv[...] * jnp.dot(t, w_ref[...], preferred_element_type=jnp.float32)


_tc2 = pl.pallas_call(
    _tc2_body,
    grid=(N1 // BR,),
    in_specs=[
        pl.BlockSpec((BR, D), lambda i: (i, 0)),
        pl.BlockSpec((BR, D), lambda i: (i, 0)),
        pl.BlockSpec((BR, D), lambda i: (i, 0)),
        pl.BlockSpec((BR, D), lambda i: (i, 0)),
        pl.BlockSpec((D, D), lambda i: (0, 0)),
        pl.BlockSpec((1, D), lambda i: (0, 0)),
    ],
    out_specs=pl.BlockSpec((BR, D), lambda i: (i, 0)),
    out_shape=jax.ShapeDtypeStruct((N1, D), jnp.float32),
)

BR3 = 2000


def _tc3_body(agg0, agg1, g2, dinv, b_ref, out_ref):
    t = dinv[...] * (agg0[...] + agg1[...] + g2[...]) + b_ref[...]
    out_ref[...] = jnp.maximum(t, 0.0)


_tc3 = pl.pallas_call(
    _tc3_body,
    grid=(N_NODES // BR3,),
    in_specs=[
        pl.BlockSpec((BR3, D), lambda i: (i, 0)),
        pl.BlockSpec((BR3, D), lambda i: (i, 0)),
        pl.BlockSpec((BR3, D), lambda i: (i, 0)),
        pl.BlockSpec((BR3, D), lambda i: (i, 0)),
        pl.BlockSpec((1, D), lambda i: (0, 0)),
    ],
    out_specs=pl.BlockSpec((BR3, D), lambda i: (i, 0)),
    out_shape=jax.ShapeDtypeStruct((N_NODES, D), jnp.float32),
)


def kernel(x, edge_index, W1, b1, W2, b2):
    src = edge_index[0].astype(jnp.int32)
    dst = edge_index[1].astype(jnp.int32)
    pad = E_PAD - N_EDGES
    fill = N_NODES + jnp.arange(pad, dtype=jnp.int32) % (N1 - N_NODES)
    src_r = jnp.concatenate([src, fill])
    dst_r = jnp.concatenate([dst, fill])
    x_pad = jnp.pad(x, ((0, N1 - N_NODES), (0, 0)))
    ones_deg = jnp.ones((DCHUNK, DEGW), jnp.float32)
    zeros_deg = jnp.zeros((ROWS_PT, DEGW), jnp.float32)
    zeros_agg = jnp.zeros((ROWS_PT, D), jnp.float32)
    b1r = b1.reshape(1, D)
    b2r = b2.reshape(1, D)

    degp = _sc_degree(dst_r, ones_deg, zeros_deg)
    dinvf, g1 = _tc1(degp[:N1], degp[N1:], x_pad, W1)
    agg1 = _sc_aggregate(g1, src_r, dst_r, zeros_agg)
    g2 = _tc2(agg1[:N1], agg1[N1:], g1, dinvf, W2, b1r)
    agg2 = _sc_aggregate(g2, src_r, dst_r, zeros_agg)
    out = _tc3(agg2[:N1], agg2[N1:], g2, dinvf, b2r)
    return out


# speedup vs baseline: 1.0208x; 1.0208x over previous
import functools

import jax
import jax.numpy as jnp
from jax import lax
from jax.experimental import pallas as pl
from jax.experimental.pallas import tpu as pltpu
from jax.experimental.pallas import tpu_sc as plsc

N_NODES = 10000
D = 128
N_EDGES = 320000

NC = 2
NS = 16
NW = NC * NS

CHUNK = 72
KCH = 144
CPW = KCH * CHUNK
E_PAD = NW * CPW
N1 = 10240
ROWS_PT = N1 // NS
DEGW = 128
DCHUNK = 128
DKCH = CPW // DCHUNK

_mesh = plsc.VectorSubcoreMesh(core_axis_name="c", subcore_axis_name="s")


@functools.partial(
    pl.kernel,
    out_type=jax.ShapeDtypeStruct((NC * N1, DEGW), jnp.float32),
    mesh=_mesh,
    scratch_types=[
        pltpu.VMEM((DCHUNK,), jnp.int32),
        pltpu.VMEM((DCHUNK,), jnp.int32),
        pltpu.VMEM((DCHUNK,), jnp.int32),
        pltpu.VMEM((DCHUNK, DEGW), jnp.float32),
        pltpu.VMEM_SHARED((N1, DEGW), jnp.float32),
        pltpu.SemaphoreType.DMA,
        pltpu.SemaphoreType.DMA,
        pltpu.SemaphoreType.DMA,
        pltpu.SemaphoreType.DMA,
        pltpu.SemaphoreType.DMA,
        pltpu.SemaphoreType.DMA,
    ],
)
def _sc_degree(dst_hbm, ones_hbm, zeros_hbm, out_hbm,
               idxd0, idxd1, idxd2, ones_v, deg_sh,
               semi0, semi1, semi2, semsc0, semsc1, semsc2):
    c = lax.axis_index("c")
    s = lax.axis_index("s")
    base = (c * NS + s) * CPW
    idxd = (idxd0, idxd1, idxd2)
    semi = (semi0, semi1, semi2)
    semsc = (semsc0, semsc1, semsc2)

    def idx_load(j, sl):
        pltpu.async_copy(
            dst_hbm.at[pl.ds(base + j * DCHUNK, DCHUNK)], idxd[sl], semi[sl])

    def idx_wait(j, sl):
        pltpu.make_async_copy(
            dst_hbm.at[pl.ds(base + j * DCHUNK, DCHUNK)], idxd[sl], semi[sl]).wait()

    def sc_issue(sl):
        pltpu.async_copy(ones_v, deg_sh.at[idxd[sl]], semsc[sl], add=True)

    def sc_wait(sl):
        pltpu.make_async_copy(ones_v, deg_sh.at[idxd[sl]], semsc[sl]).wait()

    pltpu.sync_copy(ones_hbm, ones_v)
    idx_load(0, 0)
    idx_load(1, 1)
    pltpu.sync_copy(zeros_hbm, deg_sh.at[pl.ds(s * ROWS_PT, ROWS_PT)])
    plsc.subcore_barrier()

    def body(g, carry):
        for k in range(3):
            j = 3 * g + k
            idx_wait(j, k)
            sc_issue(k)

            @pl.when(j >= 1)
            def _():
                sc_wait((k + 2) % 3)

            @pl.when(j + 2 < DKCH)
            def _():
                idx_load(j + 2, (k + 2) % 3)
        return carry

    lax.fori_loop(0, DKCH // 3, body, 0)
    sc_wait(2)
    plsc.subcore_barrier()
    pltpu.sync_copy(
        deg_sh.at[pl.ds(s * ROWS_PT, ROWS_PT)],
        out_hbm.at[pl.ds(c * N1 + s * ROWS_PT, ROWS_PT)],
    )


@functools.partial(
    pl.kernel,
    out_type=jax.ShapeDtypeStruct((NC * N1, D), jnp.float32),
    mesh=_mesh,
    scratch_types=[
        pltpu.VMEM((CHUNK,), jnp.int32),
        pltpu.VMEM((CHUNK,), jnp.int32),
        pltpu.VMEM((CHUNK,), jnp.int32),
        pltpu.VMEM((CHUNK,), jnp.int32),
        pltpu.VMEM((CHUNK,), jnp.int32),
        pltpu.VMEM((CHUNK,), jnp.int32),
        pltpu.VMEM((CHUNK,), jnp.int32),
        pltpu.VMEM((CHUNK,), jnp.int32),
        pltpu.VMEM((CHUNK,), jnp.int32),
        pltpu.VMEM((CHUNK,), jnp.int32),
        pltpu.VMEM((CHUNK,), jnp.int32),
        pltpu.VMEM((CHUNK,), jnp.int32),
        pltpu.VMEM((CHUNK, D), jnp.float32),
        pltpu.VMEM((CHUNK, D), jnp.float32),
        pltpu.VMEM((CHUNK, D), jnp.float32),
        pltpu.VMEM_SHARED((N1, D), jnp.float32),
        pltpu.SemaphoreType.DMA,
        pltpu.SemaphoreType.DMA,
        pltpu.SemaphoreType.DMA,
        pltpu.SemaphoreType.DMA,
        pltpu.SemaphoreType.DMA,
        pltpu.SemaphoreType.DMA,
        pltpu.SemaphoreType.DMA,
        pltpu.SemaphoreType.DMA,
        pltpu.SemaphoreType.DMA,
        pltpu.SemaphoreType.DMA,
        pltpu.SemaphoreType.DMA,
        pltpu.SemaphoreType.DMA,
    ],
)
def _sc_aggregate(g_hbm, src_hbm, dst_hbm, zeros_hbm, out_hbm,
                  idxs0, idxs1, idxs2, idxs3, idxs4, idxs5,
                  idxd0, idxd1, idxd2, idxd3, idxd4, idxd5,
                  rows0, rows1, rows2, agg_sh,
                  semi0, semi1, semi2, semi3, semi4, semi5,
                  semg0, semg1, semg2, semsc0, semsc1, semsc2):
    c = lax.axis_index("c")
    s = lax.axis_index("s")
    base = (c * NS + s) * CPW
    idxs = (idxs0, idxs1, idxs2, idxs3, idxs4, idxs5)
    idxd = (idxd0, idxd1, idxd2, idxd3, idxd4, idxd5)
    rows = (rows0, rows1, rows2)
    semi = (semi0, semi1, semi2, semi3, semi4, semi5)
    semg = (semg0, semg1, semg2)
    semsc = (semsc0, semsc1, semsc2)

    def idx_load(j, sl):
        pltpu.async_copy(
            src_hbm.at[pl.ds(base + j * CHUNK, CHUNK)], idxs[sl], semi[sl])
        pltpu.async_copy(
            dst_hbm.at[pl.ds(base + j * CHUNK, CHUNK)], idxd[sl], semi[sl])

    def idx_wait(j, sl):
        pltpu.make_async_copy(
            src_hbm.at[pl.ds(base + j * CHUNK, CHUNK)], idxs[sl], semi[sl]).wait()
        pltpu.make_async_copy(
            dst_hbm.at[pl.ds(base + j * CHUNK, CHUNK)], idxd[sl], semi[sl]).wait()

    def gather(sl, r):
        pltpu.async_copy(g_hbm.at[idxs[sl]], rows[r], semg[r])

    def gather_wait(sl, r):
        pltpu.make_async_copy(g_hbm.at[idxs[sl]], rows[r], semg[r]).wait()

    def sc_issue(sl, r):
        pltpu.async_copy(rows[r], agg_sh.at[idxd[sl]], semsc[r], add=True)

    def sc_wait(sl, r):
        pltpu.make_async_copy(rows[r], agg_sh.at[idxd[sl]], semsc[r]).wait()

    for k in range(5):
        idx_load(k, k)
    pltpu.sync_copy(zeros_hbm, agg_sh.at[pl.ds(s * ROWS_PT, ROWS_PT)])
    plsc.subcore_barrier()
    idx_wait(0, 0)
    gather(0, 0)
    idx_wait(1, 1)
    gather(1, 1)

    def body(g, carry):
        for k in range(6):
            j = 6 * g + k
            r = k % 3
            rm1 = (k + 2) % 3
            gather_wait(k, r)
            sc_issue(k, r)

            @pl.when(j >= 1)
            def _():
                sc_wait((k + 5) % 6, rm1)

            @pl.when(j + 5 < KCH)
            def _():
                idx_load(j + 5, (k + 5) % 6)

            @pl.when(j + 2 < KCH)
            def _():
                idx_wait(j + 2, (k + 2) % 6)
                gather((k + 2) % 6, rm1)
        return carry

    lax.fori_loop(0, KCH // 6, body, 0)
    sc_wait(5, 2)
    plsc.subcore_barrier()
    pltpu.sync_copy(
        agg_sh.at[pl.ds(s * ROWS_PT, ROWS_PT)],
        out_hbm.at[pl.ds(c * N1 + s * ROWS_PT, ROWS_PT)],
    )


BR = 1024


def _tc1_body(degp0, degp1, x_ref, w_ref, dinv_ref, g_ref):
    deg = degp0[:, :1] + degp1[:, :1] + 1.0
    dinvb = jnp.broadcast_to(lax.rsqrt(deg), (BR, D))
    h = jnp.dot(x_ref[...], w_ref[...], preferred_element_type=jnp.float32)
    dinv_ref[...] = dinvb
    g_ref[...] = dinvb * h


_tc1 = pl.pallas_call(
    _tc1_body,
    grid=(N1 // BR,),
    in_specs=[
        pl.BlockSpec((BR, DEGW), lambda i: (i, 0)),
        pl.BlockSpec((BR, DEGW), lambda i: (i, 0)),
        pl.BlockSpec((BR, D), lambda i: (i, 0)),
        pl.BlockSpec((D, D), lambda i: (0, 0)),
    ],
    out_specs=[
        pl.BlockSpec((BR, D), lambda i: (i, 0)),
        pl.BlockSpec((BR, D), lambda i: (i, 0)),
    ],
    out_shape=[
        jax.ShapeDtypeStruct((N1, D), jnp.float32),
        jax.ShapeDtypeStruct((N1, D), jnp.float32),
    ],
)


def _tc2_body(agg0, agg1, g1, dinv, w_ref, b_ref, g2_ref):
    i = pl.program_id(0)
    t = dinv[...] * (agg0[...] + agg1[...] + g1[...]) + b_ref[...]
    t = jnp.maximum(t, 0.0)
    rows = i * BR + lax.broadcasted_iota(jnp.int32, (BR, D), 0)
    t = jnp.where(rows < N_NODES, t, 0.0)
    g2_ref[...] = dinv[...] * jnp.dot(t, w_ref[...], preferred_element_type=jnp.float32)


_tc2 = pl.pallas_call(
    _tc2_body,
    grid=(N1 // BR,),
    in_specs=[
        pl.BlockSpec((BR, D), lambda i: (i, 0)),
        pl.BlockSpec((BR, D), lambda i: (i, 0)),
        pl.BlockSpec((BR, D), lambda i: (i, 0)),
        pl.BlockSpec((BR, D), lambda i: (i, 0)),
        pl.BlockSpec((D, D), lambda i: (0, 0)),
        pl.BlockSpec((1, D), lambda i: (0, 0)),
    ],
    out_specs=pl.BlockSpec((BR, D), lambda i: (i, 0)),
    out_shape=jax.ShapeDtypeStruct((N1, D), jnp.float32),
)

BR3 = 2000


def _tc3_body(agg0, agg1, g2, dinv, b_ref, out_ref):
    t = dinv[...] * (agg0[...] + agg1[...] + g2[...]) + b_ref[...]
    out_ref[...] = jnp.maximum(t, 0.0)


_tc3 = pl.pallas_call(
    _tc3_body,
    grid=(N_NODES // BR3,),
    in_specs=[
        pl.BlockSpec((BR3, D), lambda i: (i, 0)),
        pl.BlockSpec((BR3, D), lambda i: (i, 0)),
        pl.BlockSpec((BR3, D), lambda i: (i, 0)),
        pl.BlockSpec((BR3, D), lambda i: (i, 0)),
        pl.BlockSpec((1, D), lambda i: (0, 0)),
    ],
    out_specs=pl.BlockSpec((BR3, D), lambda i: (i, 0)),
    out_shape=jax.ShapeDtypeStruct((N_NODES, D), jnp.float32),
)


def kernel(x, edge_index, W1, b1, W2, b2):
    src = edge_index[0].astype(jnp.int32)
    dst = edge_index[1].astype(jnp.int32)
    pad = E_PAD - N_EDGES
    fill = N_NODES + jnp.arange(pad, dtype=jnp.int32) % (N1 - N_NODES)
    src_r = jnp.concatenate([src, fill])
    dst_r = jnp.concatenate([dst, fill])
    x_pad = jnp.pad(x, ((0, N1 - N_NODES), (0, 0)))
    ones_deg = jnp.ones((DCHUNK, DEGW), jnp.float32)
    zeros_deg = jnp.zeros((ROWS_PT, DEGW), jnp.float32)
    zeros_agg = jnp.zeros((ROWS_PT, D), jnp.float32)
    b1r = b1.reshape(1, D)
    b2r = b2.reshape(1, D)

    degp = _sc_degree(dst_r, ones_deg, zeros_deg)
    dinvf, g1 = _tc1(degp[:N1], degp[N1:], x_pad, W1)
    agg1 = _sc_aggregate(g1, src_r, dst_r, zeros_agg)
    g2 = _tc2(agg1[:N1], agg1[N1:], g1, dinvf, W2, b1r)
    agg2 = _sc_aggregate(g2, src_r, dst_r, zeros_agg)
    out = _tc3(agg2[:N1], agg2[N1:], g2, dinvf, b2r)
    return out


# speedup vs baseline: 1.0411x; 1.0199x over previous
import functools

import jax
import jax.numpy as jnp
from jax import lax
from jax.experimental import pallas as pl
from jax.experimental.pallas import tpu as pltpu
from jax.experimental.pallas import tpu_sc as plsc

N_NODES = 10000
D = 128
N_EDGES = 320000

NC = 2
NS = 16
NW = NC * NS

CHUNK = 96
KCH = 108
CPW = KCH * CHUNK
E_PAD = NW * CPW
N1 = 10240
ROWS_PT = N1 // NS
DEGW = 128
DCHUNK = 128
DKCH = CPW // DCHUNK

_mesh = plsc.VectorSubcoreMesh(core_axis_name="c", subcore_axis_name="s")


@functools.partial(
    pl.kernel,
    out_type=jax.ShapeDtypeStruct((NC * N1, DEGW), jnp.float32),
    mesh=_mesh,
    scratch_types=[
        pltpu.VMEM((DCHUNK,), jnp.int32),
        pltpu.VMEM((DCHUNK,), jnp.int32),
        pltpu.VMEM((DCHUNK,), jnp.int32),
        pltpu.VMEM((DCHUNK, DEGW), jnp.float32),
        pltpu.VMEM_SHARED((N1, DEGW), jnp.float32),
        pltpu.SemaphoreType.DMA,
        pltpu.SemaphoreType.DMA,
        pltpu.SemaphoreType.DMA,
        pltpu.SemaphoreType.DMA,
        pltpu.SemaphoreType.DMA,
        pltpu.SemaphoreType.DMA,
    ],
)
def _sc_degree(dst_hbm, ones_hbm, zeros_hbm, out_hbm,
               idxd0, idxd1, idxd2, ones_v, deg_sh,
               semi0, semi1, semi2, semsc0, semsc1, semsc2):
    c = lax.axis_index("c")
    s = lax.axis_index("s")
    base = (c * NS + s) * CPW
    idxd = (idxd0, idxd1, idxd2)
    semi = (semi0, semi1, semi2)
    semsc = (semsc0, semsc1, semsc2)

    def idx_load(j, sl):
        pltpu.async_copy(
            dst_hbm.at[pl.ds(base + j * DCHUNK, DCHUNK)], idxd[sl], semi[sl])

    def idx_wait(j, sl):
        pltpu.make_async_copy(
            dst_hbm.at[pl.ds(base + j * DCHUNK, DCHUNK)], idxd[sl], semi[sl]).wait()

    def sc_issue(sl):
        pltpu.async_copy(ones_v, deg_sh.at[idxd[sl]], semsc[sl], add=True)

    def sc_wait(sl):
        pltpu.make_async_copy(ones_v, deg_sh.at[idxd[sl]], semsc[sl]).wait()

    pltpu.sync_copy(ones_hbm, ones_v)
    idx_load(0, 0)
    idx_load(1, 1)
    pltpu.sync_copy(zeros_hbm, deg_sh.at[pl.ds(s * ROWS_PT, ROWS_PT)])
    plsc.subcore_barrier()

    def body(g, carry):
        for k in range(3):
            j = 3 * g + k
            idx_wait(j, k)
            sc_issue(k)

            @pl.when(j >= 1)
            def _():
                sc_wait((k + 2) % 3)

            @pl.when(j + 2 < DKCH)
            def _():
                idx_load(j + 2, (k + 2) % 3)
        return carry

    lax.fori_loop(0, DKCH // 3, body, 0)
    sc_wait(2)
    plsc.subcore_barrier()
    pltpu.sync_copy(
        deg_sh.at[pl.ds(s * ROWS_PT, ROWS_PT)],
        out_hbm.at[pl.ds(c * N1 + s * ROWS_PT, ROWS_PT)],
    )


@functools.partial(
    pl.kernel,
    out_type=jax.ShapeDtypeStruct((NC * N1, D), jnp.float32),
    mesh=_mesh,
    scratch_types=[
        pltpu.VMEM((CHUNK,), jnp.int32),
        pltpu.VMEM((CHUNK,), jnp.int32),
        pltpu.VMEM((CHUNK,), jnp.int32),
        pltpu.VMEM((CHUNK,), jnp.int32),
        pltpu.VMEM((CHUNK,), jnp.int32),
        pltpu.VMEM((CHUNK,), jnp.int32),
        pltpu.VMEM((CHUNK,), jnp.int32),
        pltpu.VMEM((CHUNK,), jnp.int32),
        pltpu.VMEM((CHUNK,), jnp.int32),
        pltpu.VMEM((CHUNK,), jnp.int32),
        pltpu.VMEM((CHUNK,), jnp.int32),
        pltpu.VMEM((CHUNK,), jnp.int32),
        pltpu.VMEM((CHUNK, D), jnp.float32),
        pltpu.VMEM((CHUNK, D), jnp.float32),
        pltpu.VMEM((CHUNK, D), jnp.float32),
        pltpu.VMEM_SHARED((N1, D), jnp.float32),
        pltpu.SemaphoreType.DMA,
        pltpu.SemaphoreType.DMA,
        pltpu.SemaphoreType.DMA,
        pltpu.SemaphoreType.DMA,
        pltpu.SemaphoreType.DMA,
        pltpu.SemaphoreType.DMA,
        pltpu.SemaphoreType.DMA,
        pltpu.SemaphoreType.DMA,
        pltpu.SemaphoreType.DMA,
        pltpu.SemaphoreType.DMA,
        pltpu.SemaphoreType.DMA,
        pltpu.SemaphoreType.DMA,
    ],
)
def _sc_aggregate(g_hbm, src_hbm, dst_hbm, zeros_hbm, out_hbm,
                  idxs0, idxs1, idxs2, idxs3, idxs4, idxs5,
                  idxd0, idxd1, idxd2, idxd3, idxd4, idxd5,
                  rows0, rows1, rows2, agg_sh,
                  semi0, semi1, semi2, semi3, semi4, semi5,
                  semg0, semg1, semg2, semsc0, semsc1, semsc2):
    c = lax.axis_index("c")
    s = lax.axis_index("s")
    base = (c * NS + s) * CPW
    idxs = (idxs0, idxs1, idxs2, idxs3, idxs4, idxs5)
    idxd = (idxd0, idxd1, idxd2, idxd3, idxd4, idxd5)
    rows = (rows0, rows1, rows2)
    semi = (semi0, semi1, semi2, semi3, semi4, semi5)
    semg = (semg0, semg1, semg2)
    semsc = (semsc0, semsc1, semsc2)

    def idx_load(j, sl):
        pltpu.async_copy(
            src_hbm.at[pl.ds(base + j * CHUNK, CHUNK)], idxs[sl], semi[sl])
        pltpu.async_copy(
            dst_hbm.at[pl.ds(base + j * CHUNK, CHUNK)], idxd[sl], semi[sl])

    def idx_wait(j, sl):
        pltpu.make_async_copy(
            src_hbm.at[pl.ds(base + j * CHUNK, CHUNK)], idxs[sl], semi[sl]).wait()
        pltpu.make_async_copy(
            dst_hbm.at[pl.ds(base + j * CHUNK, CHUNK)], idxd[sl], semi[sl]).wait()

    def gather(sl, r):
        pltpu.async_copy(g_hbm.at[idxs[sl]], rows[r], semg[r])

    def gather_wait(sl, r):
        pltpu.make_async_copy(g_hbm.at[idxs[sl]], rows[r], semg[r]).wait()

    def sc_issue(sl, r):
        pltpu.async_copy(rows[r], agg_sh.at[idxd[sl]], semsc[r], add=True)

    def sc_wait(sl, r):
        pltpu.make_async_copy(rows[r], agg_sh.at[idxd[sl]], semsc[r]).wait()

    for k in range(5):
        idx_load(k, k)
    pltpu.sync_copy(zeros_hbm, agg_sh.at[pl.ds(s * ROWS_PT, ROWS_PT)])
    plsc.subcore_barrier()
    idx_wait(0, 0)
    gather(0, 0)
    idx_wait(1, 1)
    gather(1, 1)

    def body(g, carry):
        for k in range(6):
            j = 6 * g + k
            r = k % 3
            rm1 = (k + 2) % 3
            gather_wait(k, r)
            sc_issue(k, r)

            @pl.when(j >= 1)
            def _():
                sc_wait((k + 5) % 6, rm1)

            @pl.when(j + 5 < KCH)
            def _():
                idx_load(j + 5, (k + 5) % 6)

            @pl.when(j + 2 < KCH)
            def _():
                idx_wait(j + 2, (k + 2) % 6)
                gather((k + 2) % 6, rm1)
        return carry

    lax.fori_loop(0, KCH // 6, body, 0)
    sc_wait(5, 2)
    plsc.subcore_barrier()
    pltpu.sync_copy(
        agg_sh.at[pl.ds(s * ROWS_PT, ROWS_PT)],
        out_hbm.at[pl.ds(c * N1 + s * ROWS_PT, ROWS_PT)],
    )


BR = 1024


def _tc1_body(degp0, degp1, x_ref, w_ref, dinv_ref, g_ref):
    deg = degp0[:, :1] + degp1[:, :1] + 1.0
    dinvb = jnp.broadcast_to(lax.rsqrt(deg), (BR, D))
    h = jnp.dot(x_ref[...], w_ref[...], preferred_element_type=jnp.float32)
    dinv_ref[...] = dinvb
    g_ref[...] = dinvb * h


_tc1 = pl.pallas_call(
    _tc1_body,
    grid=(N1 // BR,),
    in_specs=[
        pl.BlockSpec((BR, DEGW), lambda i: (i, 0)),
        pl.BlockSpec((BR, DEGW), lambda i: (i, 0)),
        pl.BlockSpec((BR, D), lambda i: (i, 0)),
        pl.BlockSpec((D, D), lambda i: (0, 0)),
    ],
    out_specs=[
        pl.BlockSpec((BR, D), lambda i: (i, 0)),
        pl.BlockSpec((BR, D), lambda i: (i, 0)),
    ],
    out_shape=[
        jax.ShapeDtypeStruct((N1, D), jnp.float32),
        jax.ShapeDtypeStruct((N1, D), jnp.float32),
    ],
)


def _tc2_body(agg0, agg1, g1, dinv, w_ref, b_ref, g2_ref):
    i = pl.program_id(0)
    t = dinv[...] * (agg0[...] + agg1[...] + g1[...]) + b_ref[...]
    t = jnp.maximum(t, 0.0)
    rows = i * BR + lax.broadcasted_iota(jnp.int32, (BR, D), 0)
    t = jnp.where(rows < N_NODES, t, 0.0)
    g2_ref[...] = dinv[...] * jnp.dot(t, w_ref[...], preferred_element_type=jnp.float32)


_tc2 = pl.pallas_call(
    _tc2_body,
    grid=(N1 // BR,),
    in_specs=[
        pl.BlockSpec((BR, D), lambda i: (i, 0)),
        pl.BlockSpec((BR, D), lambda i: (i, 0)),
        pl.BlockSpec((BR, D), lambda i: (i, 0)),
        pl.BlockSpec((BR, D), lambda i: (i, 0)),
        pl.BlockSpec((D, D), lambda i: (0, 0)),
        pl.BlockSpec((1, D), lambda i: (0, 0)),
    ],
    out_specs=pl.BlockSpec((BR, D), lambda i: (i, 0)),
    out_shape=jax.ShapeDtypeStruct((N1, D), jnp.float32),
)

BR3 = 2000


def _tc3_body(agg0, agg1, g2, dinv, b_ref, out_ref):
    t = dinv[...] * (agg0[...] + agg1[...] + g2[...]) + b_ref[...]
    out_ref[...] = jnp.maximum(t, 0.0)


_tc3 = pl.pallas_call(
    _tc3_body,
    grid=(N_NODES // BR3,),
    in_specs=[
        pl.BlockSpec((BR3, D), lambda i: (i, 0)),
        pl.BlockSpec((BR3, D), lambda i: (i, 0)),
        pl.BlockSpec((BR3, D), lambda i: (i, 0)),
        pl.BlockSpec((BR3, D), lambda i: (i, 0)),
        pl.BlockSpec((1, D), lambda i: (0, 0)),
    ],
    out_specs=pl.BlockSpec((BR3, D), lambda i: (i, 0)),
    out_shape=jax.ShapeDtypeStruct((N_NODES, D), jnp.float32),
)


def kernel(x, edge_index, W1, b1, W2, b2):
    src = edge_index[0].astype(jnp.int32)
    dst = edge_index[1].astype(jnp.int32)
    pad = E_PAD - N_EDGES
    fill = N_NODES + jnp.arange(pad, dtype=jnp.int32) % (N1 - N_NODES)
    src_r = jnp.concatenate([src, fill])
    dst_r = jnp.concatenate([dst, fill])
    x_pad = jnp.pad(x, ((0, N1 - N_NODES), (0, 0)))
    ones_deg = jnp.ones((DCHUNK, DEGW), jnp.float32)
    zeros_deg = jnp.zeros((ROWS_PT, DEGW), jnp.float32)
    zeros_agg = jnp.zeros((ROWS_PT, D), jnp.float32)
    b1r = b1.reshape(1, D)
    b2r = b2.reshape(1, D)

    degp = _sc_degree(dst_r, ones_deg, zeros_deg)
    dinvf, g1 = _tc1(degp[:N1], degp[N1:], x_pad, W1)
    agg1 = _sc_aggregate(g1, src_r, dst_r, zeros_agg)
    g2 = _tc2(agg1[:N1], agg1[N1:], g1, dinvf, W2, b1r)
    agg2 = _sc_aggregate(g2, src_r, dst_r, zeros_agg)
    out = _tc3(agg2[:N1], agg2[N1:], g2, dinvf, b2r)
    return out
